# Initial kernel scaffold; baseline (speedup 1.0000x reference)
import jax
import jax.numpy as jnp
from jax.experimental import pallas as pl


def kernel(x, item_table, brand_table, cat_table, price_W, price_b, fusion_W, fusion_b, event_table, time_W, time_b, out_W, out_b):
    raise NotImplementedError("write your pallas kernel here")


# speedup vs baseline: 2.1887x; 2.1887x over previous
import functools

import jax
import jax.numpy as jnp
from jax import lax
from jax.experimental import pallas as pl
from jax.experimental.pallas import tpu as pltpu
from jax.experimental.pallas import tpu_sc as plsc

_B = 16384
_NC, _NS = 2, 16
_NW = _NC * _NS
_BPW = _B // _NW
_CH = 128
_NCH = _BPW // _CH
_D_ITEM, _D_BRAND = 64, 16
_BLK = 2048

_HI = jax.lax.Precision.HIGHEST


def _sc_gather(item_table, brand_table, item_idx2d, brand_idx2d):
    mesh = plsc.VectorSubcoreMesh(core_axis_name="c", subcore_axis_name="s")

    @functools.partial(
        pl.kernel,
        mesh=mesh,
        compiler_params=pltpu.CompilerParams(use_tc_tiling_on_sc=False),
        out_type=[
            jax.ShapeDtypeStruct((_B, _D_ITEM), jnp.float32),
            jax.ShapeDtypeStruct((_B, _D_BRAND), jnp.float32),
        ],
        scratch_types=[
            pltpu.VMEM((_NCH, _CH), jnp.int32),
            pltpu.VMEM((_NCH, _CH), jnp.int32),
            pltpu.VMEM((_BPW, _D_ITEM), jnp.float32),
            pltpu.VMEM((_BPW, _D_BRAND), jnp.float32),
            pltpu.SemaphoreType.DMA,
        ],
    )
    def k(item_hbm, brand_hbm, ii_hbm, bi_hbm, out_i, out_b,
          ii_v, bi_v, ri_v, rb_v, sem):
        wid = lax.axis_index("s") * _NC + lax.axis_index("c")
        base = wid * _BPW
        pltpu.sync_copy(ii_hbm.at[pl.ds(wid * _NCH, _NCH)], ii_v)
        pltpu.sync_copy(bi_hbm.at[pl.ds(wid * _NCH, _NCH)], bi_v)
        copies = []
        for j in range(_NCH):
            dst = pl.ds(j * _CH, _CH)
            copies.append(pltpu.async_copy(
                item_hbm.at[ii_v.at[j]], ri_v.at[dst], sem))
            copies.append(pltpu.async_copy(
                brand_hbm.at[bi_v.at[j]], rb_v.at[dst], sem))
        for c in copies:
            c.wait()
        pltpu.sync_copy(ri_v, out_i.at[pl.ds(base, _BPW)])
        pltpu.sync_copy(rb_v, out_b.at[pl.ds(base, _BPW)])

    return k(item_table, brand_table, item_idx2d, brand_idx2d)


def _tc_body(gi_ref, gb_ref, x_ref, fW_ref, oW_ref, pW_ref, pb_ref,
             tW_ref, tb_ref, et_ref, fb_ref, ob_ref, cat_ref, out_ref):
    fW = fW_ref[...]
    oW = oW_ref[...]
    O_item = oW[:, :_D_ITEM]
    O_event = oW[:, 64:72]
    O_time = oW[:, 72:80]
    W80 = fW[:, :80]
    W_cat = fW[:, 80:96]
    W_price = fW[:, 96:112]

    M80 = lax.dot_general(O_item, W80, (((1,), (0,)), ((), ())),
                          precision=_HI, preferred_element_type=jnp.float32)
    g80 = jnp.concatenate([gi_ref[...], gb_ref[...]], axis=1)
    acc = lax.dot_general(g80, M80, (((1,), (1,)), ((), ())),
                          precision=_HI, preferred_element_type=jnp.float32)

    M_cat = lax.dot_general(O_item, W_cat, (((1,), (0,)), ((), ())),
                            precision=_HI, preferred_element_type=jnp.float32)
    C = lax.dot_general(cat_ref[...], M_cat, (((1,), (1,)), ((), ())),
                        precision=_HI, preferred_element_type=jnp.float32)
    iota24 = lax.broadcasted_iota(jnp.int32, (1, 24), 1)
    oh24 = (x_ref[:, 2:3].astype(jnp.int32) == iota24).astype(jnp.float32)
    acc += lax.dot_general(oh24, C, (((1,), (0,)), ((), ())),
                           precision=_HI, preferred_element_type=jnp.float32)

    riota = lax.broadcasted_iota(jnp.int32, (4, 8), 0)
    et0 = jnp.where(riota == 0, 0.0, et_ref[...])
    E = lax.dot_general(et0, O_event, (((1,), (1,)), ((), ())),
                        precision=_HI, preferred_element_type=jnp.float32)
    iota4 = lax.broadcasted_iota(jnp.int32, (1, 4), 1)
    oh4 = (x_ref[:, 5:6].astype(jnp.int32) == iota4).astype(jnp.float32)
    acc += lax.dot_general(oh4, E, (((1,), (0,)), ((), ())),
                           precision=_HI, preferred_element_type=jnp.float32)

    pv = lax.dot_general(W_price, pW_ref[...], (((1,), (0,)), ((), ())),
                         precision=_HI, preferred_element_type=jnp.float32)
    v_price = lax.dot_general(O_item, pv, (((1,), (0,)), ((), ())),
                              precision=_HI, preferred_element_type=jnp.float32)
    v_time = lax.dot_general(O_time, tW_ref[...], (((1,), (0,)), ((), ())),
                             precision=_HI, preferred_element_type=jnp.float32)
    cb = fb_ref[...] + lax.dot_general(W_price, pb_ref[...],
                                       (((1,), (0,)), ((), ())),
                                       precision=_HI, preferred_element_type=jnp.float32)
    cvec = (lax.dot_general(O_item, cb, (((1,), (0,)), ((), ())),
                            precision=_HI, preferred_element_type=jnp.float32)
            + lax.dot_general(O_time, tb_ref[...], (((1,), (0,)), ((), ())),
                              precision=_HI, preferred_element_type=jnp.float32))
    V3 = jnp.concatenate([v_price, v_time, cvec], axis=1)
    u3 = jnp.concatenate([x_ref[:, 3:5], jnp.ones((_BLK, 1), jnp.float32)],
                         axis=1)
    acc += lax.dot_general(u3, V3, (((1,), (1,)), ((), ())),
                           precision=_HI, preferred_element_type=jnp.float32)

    out_ref[...] = acc + ob_ref[...]


def _tc_dense(gi, gb, x, fusion_W, out_W, price_W, price_b2, time_W,
              time_b2, event_table, fusion_b2, out_b2, cat_table):
    grid = (_B // _BLK,)
    full = lambda shape: pl.BlockSpec(shape, lambda i: (0, 0))
    return pl.pallas_call(
        _tc_body,
        grid=grid,
        in_specs=[
            pl.BlockSpec((_BLK, _D_ITEM), lambda i: (i, 0)),
            pl.BlockSpec((_BLK, _D_BRAND), lambda i: (i, 0)),
            pl.BlockSpec((_BLK, 6), lambda i: (i, 0)),
            full((64, 112)),
            full((128, 80)),
            full((16, 1)),
            full((16, 1)),
            full((8, 1)),
            full((8, 1)),
            full((4, 8)),
            full((64, 1)),
            full((1, 128)),
            full((24, 16)),
        ],
        out_specs=pl.BlockSpec((_BLK, 128), lambda i: (i, 0)),
        out_shape=jax.ShapeDtypeStruct((_B, 128), jnp.float32),
    )(gi, gb, x, fusion_W, out_W, price_W, price_b2, time_W, time_b2,
      event_table, fusion_b2, out_b2, cat_table)


def kernel(x, item_table, brand_table, cat_table, price_W, price_b,
           fusion_W, fusion_b, event_table, time_W, time_b, out_W, out_b):
    item_idx = x[:, 0].astype(jnp.int32).reshape(_NW * _NCH, _CH)
    brand_idx = x[:, 1].astype(jnp.int32).reshape(_NW * _NCH, _CH)
    gi, gb = _sc_gather(item_table, brand_table, item_idx, brand_idx)
    return _tc_dense(gi, gb, x, fusion_W, out_W, price_W,
                     price_b.reshape(16, 1), time_W, time_b.reshape(8, 1),
                     event_table, fusion_b.reshape(64, 1),
                     out_b.reshape(1, 128), cat_table)


# speedup vs baseline: 3.1172x; 1.4242x over previous
import functools

import jax
import jax.numpy as jnp
from jax import lax
from jax.experimental import pallas as pl
from jax.experimental.pallas import tpu as pltpu
from jax.experimental.pallas import tpu_sc as plsc

_B = 16384
_NC, _NS, _L = 2, 16, 16
_NW = _NC * _NS
_BPW = _B // _NW
_CH = 128
_NCH = _BPW // _CH
_D_ITEM, _D_BRAND = 64, 16
_BLK = 2048

_HI = lax.Precision.HIGHEST


def _sc_gather(item_idx, brand_idx, item_table, brand_table):
    mesh = plsc.VectorSubcoreMesh(core_axis_name="c", subcore_axis_name="s")

    @functools.partial(
        pl.kernel,
        mesh=mesh,
        compiler_params=pltpu.CompilerParams(use_tc_tiling_on_sc=False),
        out_type=[
            jax.ShapeDtypeStruct((_B, _D_ITEM), jnp.float32),
            jax.ShapeDtypeStruct((_B, _D_BRAND), jnp.float32),
        ],
        scratch_types=[
            pltpu.VMEM((_BPW,), jnp.int32),
            pltpu.VMEM((_BPW,), jnp.int32),
            pltpu.VMEM((_BPW, _D_ITEM), jnp.float32),
            pltpu.VMEM((_BPW, _D_BRAND), jnp.float32),
            pltpu.SemaphoreType.DMA,
        ],
    )
    def k(ii_hbm, bi_hbm, item_hbm, brand_hbm, out_i, out_b,
          ii_v, bi_v, ri_v, rb_v, sem):
        wid = lax.axis_index("s") * _NC + lax.axis_index("c")
        base = wid * _BPW
        pltpu.sync_copy(ii_hbm.at[pl.ds(base, _BPW)], ii_v)
        pltpu.sync_copy(bi_hbm.at[pl.ds(base, _BPW)], bi_v)
        copies = []
        for j in range(_NCH):
            sl = pl.ds(j * _CH, _CH)
            copies.append(pltpu.async_copy(
                item_hbm.at[ii_v.at[sl]], ri_v.at[sl], sem))
            copies.append(pltpu.async_copy(
                brand_hbm.at[bi_v.at[sl]], rb_v.at[sl], sem))
        for c in copies:
            c.wait()
        pltpu.sync_copy(ri_v, out_i.at[pl.ds(base, _BPW)])
        pltpu.sync_copy(rb_v, out_b.at[pl.ds(base, _BPW)])

    return k(item_idx, brand_idx, item_table, brand_table)


def _prep_body(fW_ref, oW_ref, pW_ref, pb_ref, tW_ref, tb_ref, et_ref,
               fb_ref, ob_ref, cat_ref, g_ref):
    fW = fW_ref[...]
    oW = oW_ref[...]
    O_item = oW[:, :_D_ITEM]
    O_event = oW[:, 64:72]
    O_time = oW[:, 72:80]
    W80 = fW[:, :80]
    W_cat = fW[:, 80:96]
    W_price = fW[:, 96:112]

    dg = lambda a, b, dims: lax.dot_general(
        a, b, (dims, ((), ())), precision=_HI,
        preferred_element_type=jnp.float32)

    g80 = dg(W80, O_item, ((0,), (1,)))
    m_cat = dg(O_item, W_cat, ((1,), (0,)))
    g_cat = dg(cat_ref[...], m_cat, ((1,), (1,)))
    riota = lax.broadcasted_iota(jnp.int32, (4, 8), 0)
    et0 = jnp.where(riota == 0, 0.0, et_ref[...])
    g_evt = dg(et0, O_event, ((1,), (1,)))
    pv = dg(W_price, pW_ref[...], ((1,), (0,)))
    g_price = dg(pv, O_item, ((0,), (1,)))
    g_time = dg(tW_ref[...], O_time, ((0,), (1,)))
    cb = fb_ref[...] + dg(W_price, pb_ref[...], ((1,), (0,)))
    g_const = (dg(cb, O_item, ((0,), (1,)))
               + dg(tb_ref[...], O_time, ((0,), (1,)))
               + ob_ref[...])
    zero = jnp.zeros((1, 128), jnp.float32)
    g_ref[...] = jnp.concatenate(
        [g80, g_cat, g_evt, g_price, g_time, g_const, zero], axis=0)


def _tc_prep(fusion_W, out_W, price_W, price_b2, time_W, time_b2,
             event_table, fusion_b2, out_b2, cat_table):
    return pl.pallas_call(
        _prep_body,
        out_shape=jax.ShapeDtypeStruct((112, 128), jnp.float32),
    )(fusion_W, out_W, price_W, price_b2, time_W, time_b2, event_table,
      fusion_b2, out_b2, cat_table)


def _main_body(gi_ref, gb_ref, x_ref, g_ref, out_ref):
    iota24 = lax.broadcasted_iota(jnp.int32, (1, 24), 1)
    oh24 = (x_ref[:, 2:3].astype(jnp.int32) == iota24).astype(jnp.float32)
    iota4 = lax.broadcasted_iota(jnp.int32, (1, 4), 1)
    oh4 = (x_ref[:, 5:6].astype(jnp.int32) == iota4).astype(jnp.float32)
    ones = jnp.ones((_BLK, 1), jnp.float32)
    zeros = jnp.zeros((_BLK, 1), jnp.float32)
    f = jnp.concatenate(
        [gi_ref[...], gb_ref[...], oh24, oh4, x_ref[:, 3:5], ones, zeros],
        axis=1)
    out_ref[...] = lax.dot_general(
        f, g_ref[...], (((1,), (0,)), ((), ())), precision=_HI,
        preferred_element_type=jnp.float32)


def _tc_main(gi, gb, x, g):
    grid = (_B // _BLK,)
    return pl.pallas_call(
        _main_body,
        grid=grid,
        in_specs=[
            pl.BlockSpec((_BLK, _D_ITEM), lambda i: (i, 0)),
            pl.BlockSpec((_BLK, _D_BRAND), lambda i: (i, 0)),
            pl.BlockSpec((_BLK, 6), lambda i: (i, 0)),
            pl.BlockSpec((112, 128), lambda i: (0, 0)),
        ],
        out_specs=pl.BlockSpec((_BLK, 128), lambda i: (i, 0)),
        out_shape=jax.ShapeDtypeStruct((_B, 128), jnp.float32),
    )(gi, gb, x, g)


def kernel(x, item_table, brand_table, cat_table, price_W, price_b,
           fusion_W, fusion_b, event_table, time_W, time_b, out_W, out_b):
    idx2 = x[:, :2].astype(jnp.int32)
    gi, gb = _sc_gather(idx2[:, 0], idx2[:, 1], item_table, brand_table)
    g = _tc_prep(fusion_W, out_W, price_W, price_b.reshape(16, 1), time_W,
                 time_b.reshape(8, 1), event_table, fusion_b.reshape(64, 1),
                 out_b.reshape(1, 128), cat_table)
    return _tc_main(gi, gb, x, g)


# speedup vs baseline: 3.4350x; 1.1020x over previous
import functools

import jax
import jax.numpy as jnp
from jax import lax
from jax.experimental import pallas as pl
from jax.experimental.pallas import tpu as pltpu
from jax.experimental.pallas import tpu_sc as plsc

_B = 16384
_NC, _NS, _L = 2, 16, 16
_NW = _NC * _NS
_BPW = _B // _NW
_CH = 128
_NCH = _BPW // _CH
_D_ITEM, _D_BRAND = 64, 16
_BLK = 2048

_HI = lax.Precision.HIGHEST


def _sc_gather(item_idx, brand_idx, item_table, brand_table):
    mesh = plsc.VectorSubcoreMesh(core_axis_name="c", subcore_axis_name="s")

    @functools.partial(
        pl.kernel,
        mesh=mesh,
        compiler_params=pltpu.CompilerParams(use_tc_tiling_on_sc=False),
        out_type=jax.ShapeDtypeStruct((_B, 128), jnp.float32),
        scratch_types=[
            pltpu.VMEM((_BPW,), jnp.int32),
            pltpu.VMEM((_BPW,), jnp.int32),
            pltpu.VMEM((_BPW, _D_ITEM), jnp.float32),
            pltpu.VMEM((_BPW, _D_BRAND), jnp.float32),
            pltpu.SemaphoreType.DMA,
        ],
    )
    def k(ii_hbm, bi_hbm, item_hbm, brand_hbm, out,
          ii_v, bi_v, ri_v, rb_v, sem):
        wid = lax.axis_index("s") * _NC + lax.axis_index("c")
        base = wid * _BPW
        pltpu.sync_copy(ii_hbm.at[pl.ds(base, _BPW)], ii_v)
        pltpu.sync_copy(bi_hbm.at[pl.ds(base, _BPW)], bi_v)
        copies = []
        for j in range(_NCH):
            sl = pl.ds(j * _CH, _CH)
            copies.append(pltpu.async_copy(
                item_hbm.at[ii_v.at[sl]], ri_v.at[sl], sem))
            copies.append(pltpu.async_copy(
                brand_hbm.at[bi_v.at[sl]], rb_v.at[sl], sem))
        for c in copies:
            c.wait()
        pltpu.sync_copy(ri_v, out.at[pl.ds(base, _BPW), pl.ds(0, _D_ITEM)])
        pltpu.sync_copy(rb_v,
                        out.at[pl.ds(base, _BPW), pl.ds(_D_ITEM, _D_BRAND)])

    return k(item_idx, brand_idx, item_table, brand_table)


def _prep_body(fW_ref, oW_ref, pW_ref, pb_ref, tW_ref, tb_ref, et_ref,
               fb_ref, ob_ref, cat_ref, g_ref):
    fW = fW_ref[...]
    oW = oW_ref[...]
    O_item = oW[:, :_D_ITEM]
    O_event = oW[:, 64:72]
    O_time = oW[:, 72:80]
    W80 = fW[:, :80]
    W_cat = fW[:, 80:96]
    W_price = fW[:, 96:112]

    dg = lambda a, b, dims: lax.dot_general(
        a, b, (dims, ((), ())), precision=_HI,
        preferred_element_type=jnp.float32)

    g80 = dg(W80, O_item, ((0,), (1,)))
    m_cat = dg(O_item, W_cat, ((1,), (0,)))
    g_cat = dg(cat_ref[...], m_cat, ((1,), (1,)))
    riota = lax.broadcasted_iota(jnp.int32, (4, 8), 0)
    et0 = jnp.where(riota == 0, 0.0, et_ref[...])
    g_evt = dg(et0, O_event, ((1,), (1,)))
    pv = dg(W_price, pW_ref[...], ((1,), (0,)))
    g_price = dg(pv, O_item, ((0,), (1,)))
    g_time = dg(tW_ref[...], O_time, ((0,), (1,)))
    cb = fb_ref[...] + dg(W_price, pb_ref[...], ((1,), (0,)))
    g_const = (dg(cb, O_item, ((0,), (1,)))
               + dg(tb_ref[...], O_time, ((0,), (1,)))
               + ob_ref[...])
    zero = jnp.zeros((1, 128), jnp.float32)
    g_ref[...] = jnp.concatenate(
        [g80, g_cat, g_evt, g_price, g_time, g_const, zero], axis=0)


def _tc_prep(fusion_W, out_W, price_W, price_b2, time_W, time_b2,
             event_table, fusion_b2, out_b2, cat_table):
    return pl.pallas_call(
        _prep_body,
        out_shape=jax.ShapeDtypeStruct((112, 128), jnp.float32),
    )(fusion_W, out_W, price_W, price_b2, time_W, time_b2, event_table,
      fusion_b2, out_b2, cat_table)


def _main_body(gf_ref, x_ref, g_ref, out_ref):
    g80 = gf_ref[:, :80]
    iota24 = lax.broadcasted_iota(jnp.int32, (1, 24), 1)
    oh24 = (x_ref[:, 2:3].astype(jnp.int32) == iota24).astype(jnp.float32)
    iota4 = lax.broadcasted_iota(jnp.int32, (1, 4), 1)
    oh4 = (x_ref[:, 5:6].astype(jnp.int32) == iota4).astype(jnp.float32)
    ones = jnp.ones((_BLK, 1), jnp.float32)
    zeros = jnp.zeros((_BLK, 1), jnp.float32)
    extras = jnp.concatenate(
        [oh24, oh4, x_ref[:, 3:5], ones, zeros], axis=1)
    g = g_ref[...]
    acc = lax.dot_general(g80, g[:80], (((1,), (0,)), ((), ())),
                          precision=_HI, preferred_element_type=jnp.float32)
    acc += lax.dot_general(extras, g[80:112], (((1,), (0,)), ((), ())),
                           precision=_HI, preferred_element_type=jnp.float32)
    out_ref[...] = acc


def _tc_main(gf, x, g):
    grid = (_B // _BLK,)
    return pl.pallas_call(
        _main_body,
        grid=grid,
        in_specs=[
            pl.BlockSpec((_BLK, 128), lambda i: (i, 0)),
            pl.BlockSpec((_BLK, 6), lambda i: (i, 0)),
            pl.BlockSpec((112, 128), lambda i: (0, 0)),
        ],
        out_specs=pl.BlockSpec((_BLK, 128), lambda i: (i, 0)),
        out_shape=jax.ShapeDtypeStruct((_B, 128), jnp.float32),
    )(gf, x, g)


def kernel(x, item_table, brand_table, cat_table, price_W, price_b,
           fusion_W, fusion_b, event_table, time_W, time_b, out_W, out_b):
    ii = x[:, 0].astype(jnp.int32)
    bi = x[:, 1].astype(jnp.int32)
    gf = _sc_gather(ii, bi, item_table, brand_table)
    g = _tc_prep(fusion_W, out_W, price_W, price_b.reshape(16, 1), time_W,
                 time_b.reshape(8, 1), event_table, fusion_b.reshape(64, 1),
                 out_b.reshape(1, 128), cat_table)
    return _tc_main(gf, x, g)


# speedup vs baseline: 4.0375x; 1.1754x over previous
import functools

import jax
import jax.numpy as jnp
from jax import lax
from jax.experimental import pallas as pl
from jax.experimental.pallas import tpu as pltpu
from jax.experimental.pallas import tpu_sc as plsc

_B = 16384
_NC, _NS, _L = 2, 16, 16
_NW = _NC * _NS
_BPW = _B // _NW
_CH = 128
_NCH = _BPW // _CH
_D_ITEM, _D_BRAND = 64, 16
_BLK = 2048

_HI = lax.Precision.HIGHEST


def _sc_gather(item_idx, brand_idx, item_table, brand_table):
    mesh = plsc.VectorSubcoreMesh(core_axis_name="c", subcore_axis_name="s")

    @functools.partial(
        pl.kernel,
        mesh=mesh,
        compiler_params=pltpu.CompilerParams(use_tc_tiling_on_sc=False),
        out_type=jax.ShapeDtypeStruct((_B, 128), jnp.float32),
        scratch_types=[
            pltpu.VMEM((_BPW,), jnp.int32),
            pltpu.VMEM((_BPW,), jnp.int32),
            pltpu.VMEM((_BPW, _D_ITEM), jnp.float32),
            pltpu.VMEM((_BPW, _D_BRAND), jnp.float32),
            pltpu.SemaphoreType.DMA,
        ],
    )
    def k(ii_hbm, bi_hbm, item_hbm, brand_hbm, out,
          ii_v, bi_v, ri_v, rb_v, sem):
        wid = lax.axis_index("s") * _NC + lax.axis_index("c")
        base = wid * _BPW
        pltpu.sync_copy(ii_hbm.at[pl.ds(base, _BPW)], ii_v)
        pltpu.sync_copy(bi_hbm.at[pl.ds(base, _BPW)], bi_v)
        copies = []
        for j in range(_NCH):
            sl = pl.ds(j * _CH, _CH)
            copies.append(pltpu.async_copy(
                item_hbm.at[ii_v.at[sl]], ri_v.at[sl], sem))
            copies.append(pltpu.async_copy(
                brand_hbm.at[bi_v.at[sl]], rb_v.at[sl], sem))
        for c in copies:
            c.wait()
        pltpu.sync_copy(ri_v, out.at[pl.ds(base, _BPW), pl.ds(0, _D_ITEM)])
        pltpu.sync_copy(rb_v,
                        out.at[pl.ds(base, _BPW), pl.ds(_D_ITEM, _D_BRAND)])

    return k(item_idx, brand_idx, item_table, brand_table)


def _prep_body(fW_ref, oW_ref, pW_ref, pb_ref, tW_ref, tb_ref, et_ref,
               fb_ref, ob_ref, cat_ref, g_ref):
    fW = fW_ref[...]
    oW = oW_ref[...]
    O_item = oW[:, :_D_ITEM]
    O_event = oW[:, 64:72]
    O_time = oW[:, 72:80]
    W80 = fW[:, :80]
    W_cat = fW[:, 80:96]
    W_price = fW[:, 96:112]

    dg = lambda a, b, dims: lax.dot_general(
        a, b, (dims, ((), ())), precision=_HI,
        preferred_element_type=jnp.float32)

    g80 = dg(W80, O_item, ((0,), (1,)))
    m_cat = dg(O_item, W_cat, ((1,), (0,)))
    g_cat = dg(cat_ref[...], m_cat, ((1,), (1,)))
    riota = lax.broadcasted_iota(jnp.int32, (4, 8), 0)
    et0 = jnp.where(riota == 0, 0.0, et_ref[...])
    g_evt = dg(et0, O_event, ((1,), (1,)))
    pv = dg(W_price, pW_ref[...], ((1,), (0,)))
    g_price = dg(pv, O_item, ((0,), (1,)))
    g_time = dg(tW_ref[...], O_time, ((0,), (1,)))
    cb = fb_ref[...] + dg(W_price, pb_ref[...], ((1,), (0,)))
    g_const = (dg(cb, O_item, ((0,), (1,)))
               + dg(tb_ref[...], O_time, ((0,), (1,)))
               + ob_ref[...])
    zero = jnp.zeros((1, 128), jnp.float32)
    g_ref[...] = jnp.concatenate(
        [g80, g_cat, g_evt, g_price, g_time, g_const, zero], axis=0)


def _tc_prep(fusion_W, out_W, price_W, price_b2, time_W, time_b2,
             event_table, fusion_b2, out_b2, cat_table):
    return pl.pallas_call(
        _prep_body,
        out_shape=jax.ShapeDtypeStruct((112, 128), jnp.float32),
    )(fusion_W, out_W, price_W, price_b2, time_W, time_b2, event_table,
      fusion_b2, out_b2, cat_table)


def _main_body(gf_ref, xt_ref, g_ref, out_ref):
    g80 = gf_ref[:, :80]
    iota24 = lax.broadcasted_iota(jnp.int32, (24, 1), 0)
    oh24 = (xt_ref[2:3, :].astype(jnp.int32) == iota24).astype(jnp.float32)
    iota4 = lax.broadcasted_iota(jnp.int32, (4, 1), 0)
    oh4 = (xt_ref[5:6, :].astype(jnp.int32) == iota4).astype(jnp.float32)
    ones = jnp.ones((1, _BLK), jnp.float32)
    zeros = jnp.zeros((1, _BLK), jnp.float32)
    extras_t = jnp.concatenate(
        [oh24, oh4, xt_ref[3:5, :], ones, zeros], axis=0)
    g = g_ref[...]
    acc = lax.dot_general(g80, g[:80], (((1,), (0,)), ((), ())),
                          precision=_HI, preferred_element_type=jnp.float32)
    acc += lax.dot_general(extras_t, g[80:112], (((0,), (0,)), ((), ())),
                           precision=_HI, preferred_element_type=jnp.float32)
    out_ref[...] = acc


def _tc_main(gf, xt, g):
    grid = (_B // _BLK,)
    return pl.pallas_call(
        _main_body,
        grid=grid,
        in_specs=[
            pl.BlockSpec((_BLK, 128), lambda i: (i, 0)),
            pl.BlockSpec((6, _BLK), lambda i: (0, i)),
            pl.BlockSpec((112, 128), lambda i: (0, 0)),
        ],
        out_specs=pl.BlockSpec((_BLK, 128), lambda i: (i, 0)),
        out_shape=jax.ShapeDtypeStruct((_B, 128), jnp.float32),
    )(gf, xt, g)


def kernel(x, item_table, brand_table, cat_table, price_W, price_b,
           fusion_W, fusion_b, event_table, time_W, time_b, out_W, out_b):
    xt = x.T
    ii = xt[0].astype(jnp.int32)
    bi = xt[1].astype(jnp.int32)
    gf = _sc_gather(ii, bi, item_table, brand_table)
    g = _tc_prep(fusion_W, out_W, price_W, price_b.reshape(16, 1), time_W,
                 time_b.reshape(8, 1), event_table, fusion_b.reshape(64, 1),
                 out_b.reshape(1, 128), cat_table)
    return _tc_main(gf, xt, g)


# speedup vs baseline: 4.0475x; 1.0025x over previous
import functools

import jax
import jax.numpy as jnp
from jax import lax
from jax.experimental import pallas as pl
from jax.experimental.pallas import tpu as pltpu
from jax.experimental.pallas import tpu_sc as plsc

_B = 16384
_NC, _NS, _L = 2, 16, 16
_NW = _NC * _NS
_BPW = _B // _NW
_CH = 128
_NCH = _BPW // _CH
_D_ITEM, _D_BRAND = 64, 16
_BLK = 4096

_HI = lax.Precision.HIGHEST


def _sc_gather(item_idx, brand_idx, item_table, brand_table):
    mesh = plsc.VectorSubcoreMesh(core_axis_name="c", subcore_axis_name="s")

    @functools.partial(
        pl.kernel,
        mesh=mesh,
        compiler_params=pltpu.CompilerParams(use_tc_tiling_on_sc=False),
        out_type=jax.ShapeDtypeStruct((_B, 128), jnp.float32),
        scratch_types=[
            pltpu.VMEM((_BPW,), jnp.int32),
            pltpu.VMEM((_BPW,), jnp.int32),
            pltpu.VMEM((_BPW, _D_ITEM), jnp.float32),
            pltpu.VMEM((_BPW, _D_BRAND), jnp.float32),
            pltpu.SemaphoreType.DMA,
        ],
    )
    def k(ii_hbm, bi_hbm, item_hbm, brand_hbm, out,
          ii_v, bi_v, ri_v, rb_v, sem):
        wid = lax.axis_index("s") * _NC + lax.axis_index("c")
        base = wid * _BPW
        pltpu.sync_copy(ii_hbm.at[pl.ds(base, _BPW)], ii_v)
        pltpu.sync_copy(bi_hbm.at[pl.ds(base, _BPW)], bi_v)
        copies = []
        for j in range(_NCH):
            sl = pl.ds(j * _CH, _CH)
            copies.append(pltpu.async_copy(
                item_hbm.at[ii_v.at[sl]], ri_v.at[sl], sem))
            copies.append(pltpu.async_copy(
                brand_hbm.at[bi_v.at[sl]], rb_v.at[sl], sem))
        for c in copies:
            c.wait()
        pltpu.sync_copy(ri_v, out.at[pl.ds(base, _BPW), pl.ds(0, _D_ITEM)])
        pltpu.sync_copy(rb_v,
                        out.at[pl.ds(base, _BPW), pl.ds(_D_ITEM, _D_BRAND)])

    return k(item_idx, brand_idx, item_table, brand_table)


def _prep_body(fW_ref, oW_ref, pW_ref, pb_ref, tW_ref, tb_ref, et_ref,
               fb_ref, ob_ref, cat_ref, g_ref):
    fW = fW_ref[...]
    oW = oW_ref[...]
    O_item = oW[:, :_D_ITEM]
    O_event = oW[:, 64:72]
    O_time = oW[:, 72:80]
    W80 = fW[:, :80]
    W_cat = fW[:, 80:96]
    W_price = fW[:, 96:112]

    dg = lambda a, b, dims: lax.dot_general(
        a, b, (dims, ((), ())), precision=_HI,
        preferred_element_type=jnp.float32)

    g80 = dg(W80, O_item, ((0,), (1,)))
    m_cat = dg(O_item, W_cat, ((1,), (0,)))
    g_cat = dg(cat_ref[...], m_cat, ((1,), (1,)))
    riota = lax.broadcasted_iota(jnp.int32, (4, 8), 0)
    et0 = jnp.where(riota == 0, 0.0, et_ref[...])
    g_evt = dg(et0, O_event, ((1,), (1,)))
    pv = dg(W_price, pW_ref[...], ((1,), (0,)))
    g_price = dg(pv, O_item, ((0,), (1,)))
    g_time = dg(tW_ref[...], O_time, ((0,), (1,)))
    cb = fb_ref[...] + dg(W_price, pb_ref[...], ((1,), (0,)))
    g_const = (dg(cb, O_item, ((0,), (1,)))
               + dg(tb_ref[...], O_time, ((0,), (1,)))
               + ob_ref[...])
    zero = jnp.zeros((1, 128), jnp.float32)
    g_ref[...] = jnp.concatenate(
        [g80, g_cat, g_evt, g_price, g_time, g_const, zero], axis=0)


def _tc_prep(fusion_W, out_W, price_W, price_b2, time_W, time_b2,
             event_table, fusion_b2, out_b2, cat_table):
    return pl.pallas_call(
        _prep_body,
        out_shape=jax.ShapeDtypeStruct((112, 128), jnp.float32),
    )(fusion_W, out_W, price_W, price_b2, time_W, time_b2, event_table,
      fusion_b2, out_b2, cat_table)


def _main_body(gf_ref, xt_ref, g_ref, out_ref):
    g80 = gf_ref[:, :80]
    iota24 = lax.broadcasted_iota(jnp.int32, (24, 1), 0)
    oh24 = (xt_ref[2:3, :].astype(jnp.int32) == iota24).astype(jnp.float32)
    iota4 = lax.broadcasted_iota(jnp.int32, (4, 1), 0)
    oh4 = (xt_ref[5:6, :].astype(jnp.int32) == iota4).astype(jnp.float32)
    ones = jnp.ones((1, _BLK), jnp.float32)
    zeros = jnp.zeros((1, _BLK), jnp.float32)
    extras_t = jnp.concatenate(
        [oh24, oh4, xt_ref[3:5, :], ones, zeros], axis=0)
    g = g_ref[...]
    acc = lax.dot_general(g80, g[:80], (((1,), (0,)), ((), ())),
                          precision=_HI, preferred_element_type=jnp.float32)
    acc += lax.dot_general(extras_t, g[80:112], (((0,), (0,)), ((), ())),
                           precision=_HI, preferred_element_type=jnp.float32)
    out_ref[...] = acc


def _tc_main(gf, xt, g):
    grid = (_B // _BLK,)
    return pl.pallas_call(
        _main_body,
        grid=grid,
        in_specs=[
            pl.BlockSpec((_BLK, 128), lambda i: (i, 0)),
            pl.BlockSpec((6, _BLK), lambda i: (0, i)),
            pl.BlockSpec((112, 128), lambda i: (0, 0)),
        ],
        out_specs=pl.BlockSpec((_BLK, 128), lambda i: (i, 0)),
        out_shape=jax.ShapeDtypeStruct((_B, 128), jnp.float32),
    )(gf, xt, g)


def kernel(x, item_table, brand_table, cat_table, price_W, price_b,
           fusion_W, fusion_b, event_table, time_W, time_b, out_W, out_b):
    xt = x.T
    ii = x[:, 0].astype(jnp.int32)
    bi = x[:, 1].astype(jnp.int32)
    gf = _sc_gather(ii, bi, item_table, brand_table)
    g = _tc_prep(fusion_W, out_W, price_W, price_b.reshape(16, 1), time_W,
                 time_b.reshape(8, 1), event_table, fusion_b.reshape(64, 1),
                 out_b.reshape(1, 128), cat_table)
    return _tc_main(gf, xt, g)
